# baseline (speedup 1.0000x reference)
import functools

import jax
import jax.numpy as jnp
from jax import lax
from jax.experimental import pallas as pl
from jax.experimental.pallas import tpu as pltpu
from jax.experimental.pallas import tpu_sc as plsc

_NC = 2
_NS = 16
_NW = _NC * _NS
_L = 16
_D = 64
_SL = 8
_B = 16384
_BPW = _B // _NW
_QB = _BPW // 4


def _sc_body(i_hbm, j_hbm, te_hbm, ce_hbm, tb_hbm, cb_hbm, out_hbm,
             iv, jv, tv0, cv0, tv1, cv1, tbv, cbv, sv,
             semt0, semc0, semt1, semc1, semb):
    wid = lax.axis_index("s") * _NC + lax.axis_index("c")
    base = wid * _BPW
    pltpu.sync_copy(i_hbm.at[wid], iv)
    pltpu.sync_copy(j_hbm.at[wid], jv)

    bias_copies = []
    for blk in range(4):
        r = pl.ds(blk * 128, 128)
        bias_copies.append(pltpu.async_copy(tb_hbm.at[iv.at[blk]], tbv.at[r], semb))
        bias_copies.append(pltpu.async_copy(cb_hbm.at[jv.at[blk]], cbv.at[r], semb))

    lanes = lax.iota(jnp.int32, _L)

    def fire_quarter(q, tbuf, cbuf, st, sc_):
        def fire(g, carry):
            rr = q * _QB + g * _L
            blk = lax.shift_right_logical(rr, 7)
            col = jnp.bitwise_and(rr, 127)
            ivec = iv[blk, pl.ds(col, _L)]
            jvec = jv[blk, pl.ds(col, _L)]
            for r in range(_L):
                pltpu.async_copy(te_hbm.at[ivec[r]], tbuf.at[g * _L + r], st)
                pltpu.async_copy(ce_hbm.at[jvec[r]], cbuf.at[g * _L + r], sc_)
            return carry

        lax.fori_loop(0, _QB // _L, fire, 0, unroll=False)

    def drain_quarter(tbuf, cbuf, st, sc_):
        pltpu.make_async_copy(te_hbm.at[pl.ds(0, _QB)], tbuf, st).wait()
        pltpu.make_async_copy(ce_hbm.at[pl.ds(0, _QB)], cbuf, sc_).wait()

    def compute_quarter(q, tbuf, cbuf):
        def group(g, carry):
            o = g * _L
            svec = jnp.zeros((_L,), jnp.float32)
            for r in range(_L):
                acc = tbuf[o + r, pl.ds(0, _L)] * cbuf[o + r, pl.ds(0, _L)]
                for c in range(1, _D // _L):
                    acc = acc + (tbuf[o + r, pl.ds(c * _L, _L)]
                                 * cbuf[o + r, pl.ds(c * _L, _L)])
                svec = jnp.where(lanes == r, jnp.sum(acc), svec)
            sv[pl.ds(q * _QB + o, _L)] = svec
            return carry

        lax.fori_loop(0, _QB // _L, group, 0, unroll=False)

    bufs = ((tv0, cv0, semt0, semc0), (tv1, cv1, semt1, semc1))
    fire_quarter(0, *bufs[0])
    for q in range(1, _BPW // _QB):
        fire_quarter(q, *bufs[q % 2])
        drain_quarter(*bufs[(q - 1) % 2])
        compute_quarter(q - 1, bufs[(q - 1) % 2][0], bufs[(q - 1) % 2][1])
    last = _BPW // _QB - 1
    drain_quarter(*bufs[last % 2])
    compute_quarter(last, bufs[last % 2][0], bufs[last % 2][1])

    for c in bias_copies:
        c.wait()
    for g in range(_BPW // _L):
        o = g * _L
        sv[pl.ds(o, _L)] = sv[pl.ds(o, _L)] + tbv[pl.ds(o, _L)] + cbv[pl.ds(o, _L)]
    pltpu.sync_copy(sv, out_hbm.at[pl.ds(base, _BPW)])


@functools.lru_cache(maxsize=1)
def _sc_gather_dot():
    mesh = plsc.VectorSubcoreMesh(core_axis_name="c", subcore_axis_name="s")
    return functools.partial(
        pl.kernel, mesh=mesh,
        compiler_params=pltpu.CompilerParams(needs_layout_passes=False),
        out_type=jax.ShapeDtypeStruct((_B,), jnp.float32),
        scratch_types=[
            pltpu.VMEM((4, 128), jnp.int32),
            pltpu.VMEM((4, 128), jnp.int32),
            pltpu.VMEM((_QB, _D), jnp.float32),
            pltpu.VMEM((_QB, _D), jnp.float32),
            pltpu.VMEM((_QB, _D), jnp.float32),
            pltpu.VMEM((_QB, _D), jnp.float32),
            pltpu.VMEM((_BPW,), jnp.float32),
            pltpu.VMEM((_BPW,), jnp.float32),
            pltpu.VMEM((_BPW,), jnp.float32),
            pltpu.SemaphoreType.DMA,
            pltpu.SemaphoreType.DMA,
            pltpu.SemaphoreType.DMA,
            pltpu.SemaphoreType.DMA,
            pltpu.SemaphoreType.DMA,
        ],
    )(_sc_body)


def _tc_cost_body(s_ref, co_ref, out_ref):
    s = s_ref[...]
    co = co_ref[...]
    w = jnp.minimum(1.0, jnp.exp(0.75 * jnp.log(co * (1.0 / 100.0))))
    e = s - jnp.log(co + 1.0)
    out_ref[0, 0] = jnp.sum(w * e * e)


def _tc_cost(s, co):
    out = pl.pallas_call(
        _tc_cost_body,
        out_shape=jax.ShapeDtypeStruct((1, 1), jnp.float32),
        out_specs=pl.BlockSpec(memory_space=pltpu.SMEM),
    )(s.reshape(128, 128), co.reshape(128, 128))
    return out[0, 0]


def kernel(i_ids, j_ids, co_occurs, target_embeddings, context_embeddings,
           target_biases, context_biases):
    i3 = i_ids.astype(jnp.int32).reshape(_NW, 4, 128)
    j3 = j_ids.astype(jnp.int32).reshape(_NW, 4, 128)
    s = _sc_gather_dot()(i3, j3, target_embeddings, context_embeddings,
                         target_biases, context_biases)
    return _tc_cost(s, co_occurs)


# speedup vs baseline: 1.9985x; 1.9985x over previous
import functools

import jax
import jax.numpy as jnp
from jax import lax
from jax.experimental import pallas as pl
from jax.experimental.pallas import tpu as pltpu
from jax.experimental.pallas import tpu_sc as plsc

_NC = 2
_NS = 16
_NW = _NC * _NS
_L = 16
_D = 64
_B = 16384
_BPW = _B // _NW
_V = 1000000
_NBLK = (_V + 127) // 128
_FULL = _V // 128
_NID = 2 * _B


def _ids_chunk(idv, c):
    return idv[lax.shift_right_logical(c, 3),
               pl.ds(jnp.bitwise_and(c, 7) * _L, _L)]


def _sc_sweep_body(ids_hbm, te_hbm, ce_hbm, tet_hbm, cet_hbm, dum_hbm,
                   out_hbm,
                   idv, hits, counts, offs, cur,
                   sat, sac, sbt, sbc, tt, tc_, stag,
                   semA, semB, semw0, semw1, semw2, semw3):
    wid = lax.axis_index("s") * _NC + lax.axis_index("c")
    lo = lax.shift_right_logical(wid * _NBLK, 5)
    hi = lax.shift_right_logical((wid + 1) * _NBLK, 5)
    pltpu.sync_copy(ids_hbm, idv)

    lanes = lax.iota(jnp.int32, _L)
    zeros = jnp.zeros((_L,), jnp.int32)
    ones = jnp.full((_L,), 1, jnp.int32)

    for k in range(256 // _L):
        counts[pl.ds(k * _L, _L)] = zeros

    def hist(c, carry):
        v = _ids_chunk(idv, c)
        vb = lax.shift_right_logical(v, 7)
        mask = jnp.logical_and(vb >= lo, vb < hi)
        rel = jnp.clip(vb - lo, 0, 255)
        plsc.addupdate_scatter(counts, [rel], ones, mask=mask)
        return carry

    lax.fori_loop(0, _NID // _L, hist, 0, unroll=False)

    run = jnp.int32(0)
    for k in range(256 // _L):
        sl = pl.ds(k * _L, _L)
        ch = counts[sl]
        inc = plsc.cumsum(ch)
        offs[sl] = run + inc - ch
        cur[sl] = run + inc - ch
        run = run + inc[15]

    def place(c, carry):
        v = _ids_chunk(idv, c)
        vb = lax.shift_right_logical(v, 7)
        mask = jnp.logical_and(vb >= lo, vb < hi)
        rel = jnp.clip(vb - lo, 0, 255)
        prior = plsc.load_gather(cur, [rel])
        occ = plsc.scan_count(rel, mask=mask)[0]
        b = c * _L + lanes
        payload = jnp.bitwise_or(jnp.bitwise_and(v, 127),
                                 lax.shift_left(b, 7))
        pos = jnp.clip(prior + occ - 1, 0, _NID - 1)
        plsc.store_scatter(hits, [pos], payload, mask=mask)
        plsc.addupdate_scatter(cur, [rel], ones, mask=mask)
        return carry

    lax.fori_loop(0, _NID // _L, place, 0, unroll=False)

    dchunks = [k * _L + lanes for k in range(_D // _L)]

    def extract_hits(rel, tbuf, cbuf, semw_list):
        s0 = plsc.load_gather(offs, [jnp.full((_L,), rel, jnp.int32)])[0]
        n0 = plsc.load_gather(counts, [jnp.full((_L,), rel, jnp.int32)])[0]

        def hit_body(h, carry):
            hv = plsc.load_gather(hits, [jnp.full((_L,), h, jnp.int32)])[0]
            lane = jnp.bitwise_and(hv, 127)
            b = lax.shift_right_logical(hv, 7)
            tb1 = lax.shift_right_logical(b, 14)
            for slot in range(4):
                @pl.when(jnp.bitwise_and(h, 3) == slot)
                def _():
                    @pl.when(h >= 4)
                    def _():
                        pltpu.make_async_copy(
                            dum_hbm, stag.at[pl.ds(0, _D)],
                            semw_list[slot]).wait()
                    ssl = pl.ds(slot * _D, _D)
                    for tsel in range(2):
                        @pl.when(tb1 == tsel)
                        def _():
                            src = tbuf if tsel == 0 else cbuf
                            lanev = jnp.full((_L,), lane, jnp.int32)
                            for k in range(_D // _L):
                                stag[pl.ds(slot * _D + k * _L, _L)] = (
                                    plsc.load_gather(src, [dchunks[k], lanev]))
                    pltpu.async_copy(stag.at[ssl],
                                     out_hbm.at[pl.ds(b * _D, _D)],
                                     semw_list[slot])
            return carry

        lax.fori_loop(s0, s0 + n0, hit_body, 0, unroll=False)

    himain = jnp.minimum(hi, _FULL)

    def fire(vb, tbuf, cbuf, sem):
        sl = pl.ds(vb * 128, 128)
        pltpu.async_copy(te_hbm.at[:, sl], tbuf, sem)
        pltpu.async_copy(ce_hbm.at[:, sl], cbuf, sem)

    def drain(tbuf, cbuf, sem):
        pltpu.make_async_copy(te_hbm.at[:, pl.ds(0, 128)], tbuf, sem).wait()
        pltpu.make_async_copy(ce_hbm.at[:, pl.ds(0, 128)], cbuf, sem).wait()

    @pl.when(lo < himain)
    def _():
        fire(lo, sat, sac, semA)

    @pl.when(lo + 1 < himain)
    def _():
        fire(lo + 1, sbt, sbc, semB)

    npairs = lax.shift_right_logical(himain - lo + 1, 1)

    def pair_body(k, carry):
        vb0 = lo + 2 * k
        vb1 = vb0 + 1

        @pl.when(vb0 < himain)
        def _():
            drain(sat, sac, semA)
            extract_hits(vb0 - lo, sat, sac, [semw0, semw1, semw2, semw3])

        @pl.when(vb0 + 2 < himain)
        def _():
            fire(vb0 + 2, sat, sac, semA)

        @pl.when(vb1 < himain)
        def _():
            drain(sbt, sbc, semB)
            extract_hits(vb1 - lo, sbt, sbc, [semw0, semw1, semw2, semw3])

        @pl.when(vb1 + 2 < himain)
        def _():
            fire(vb1 + 2, sbt, sbc, semB)

        return carry

    lax.fori_loop(0, npairs, pair_body, 0, unroll=False)

    @pl.when(hi == _NBLK)
    def _():
        pltpu.sync_copy(tet_hbm, tt)
        pltpu.sync_copy(cet_hbm, tc_)
        extract_hits(_FULL - lo, tt, tc_, [semw0, semw1, semw2, semw3])

    relL = jnp.full((_L,), hi - lo - 1, jnp.int32)
    nh = plsc.load_gather(offs, [relL])[0] + plsc.load_gather(counts, [relL])[0]
    for slot, sem in enumerate((semw0, semw1, semw2, semw3)):
        @pl.when(nh > slot)
        def _():
            pltpu.make_async_copy(dum_hbm, stag.at[pl.ds(0, _D)], sem).wait()


@functools.lru_cache(maxsize=1)
def _sc_sweep():
    mesh = plsc.VectorSubcoreMesh(core_axis_name="c", subcore_axis_name="s")
    return functools.partial(
        pl.kernel, mesh=mesh,
        compiler_params=pltpu.CompilerParams(needs_layout_passes=False),
        out_type=jax.ShapeDtypeStruct((_NID * _D,), jnp.float32),
        scratch_types=[
            pltpu.VMEM((_NID // 128, 128), jnp.int32),
            pltpu.VMEM((_NID,), jnp.int32),
            pltpu.VMEM((256,), jnp.int32),
            pltpu.VMEM((256,), jnp.int32),
            pltpu.VMEM((256,), jnp.int32),
            pltpu.VMEM((_D, 128), jnp.float32),
            pltpu.VMEM((_D, 128), jnp.float32),
            pltpu.VMEM((_D, 128), jnp.float32),
            pltpu.VMEM((_D, 128), jnp.float32),
            pltpu.VMEM((_D, 64), jnp.float32),
            pltpu.VMEM((_D, 64), jnp.float32),
            pltpu.VMEM((4 * _D,), jnp.float32),
            pltpu.SemaphoreType.DMA,
            pltpu.SemaphoreType.DMA,
            pltpu.SemaphoreType.DMA,
            pltpu.SemaphoreType.DMA,
            pltpu.SemaphoreType.DMA,
            pltpu.SemaphoreType.DMA,
        ],
    )(_sc_sweep_body)


def _sc_dot_body(i_hbm, j_hbm, rows_hbm, tb_hbm, cb_hbm, out_hbm,
                 iv, jv, tv, cv, tbv, cbv, sv, semr, semb):
    wid = lax.axis_index("s") * _NC + lax.axis_index("c")
    base = wid * _BPW
    pltpu.sync_copy(i_hbm.at[wid], iv)
    pltpu.sync_copy(j_hbm.at[wid], jv)

    bias_copies = []
    for blk in range(4):
        r = pl.ds(blk * 128, 128)
        bias_copies.append(pltpu.async_copy(tb_hbm.at[iv.at[blk]], tbv.at[r], semb))
        bias_copies.append(pltpu.async_copy(cb_hbm.at[jv.at[blk]], cbv.at[r], semb))

    ct = pltpu.async_copy(rows_hbm.at[pl.ds(base * _D, _BPW * _D)], tv, semr)
    cc = pltpu.async_copy(
        rows_hbm.at[pl.ds((_B + base) * _D, _BPW * _D)], cv, semr)
    ct.wait()
    cc.wait()
    for c in bias_copies:
        c.wait()

    lanes = lax.iota(jnp.int32, _L)

    def group(g, carry):
        o = g * _L
        svec = jnp.zeros((_L,), jnp.float32)
        for r in range(_L):
            rb = (o + r) * _D
            acc = tv[pl.ds(rb, _L)] * cv[pl.ds(rb, _L)]
            for c in range(1, _D // _L):
                acc = acc + (tv[pl.ds(rb + c * _L, _L)]
                             * cv[pl.ds(rb + c * _L, _L)])
            svec = jnp.where(lanes == r, jnp.sum(acc), svec)
        sl = pl.ds(o, _L)
        sv[sl] = svec + tbv[sl] + cbv[sl]
        return carry

    lax.fori_loop(0, _BPW // _L, group, 0, unroll=False)
    pltpu.sync_copy(sv, out_hbm.at[pl.ds(base, _BPW)])


@functools.lru_cache(maxsize=1)
def _sc_dot():
    mesh = plsc.VectorSubcoreMesh(core_axis_name="c", subcore_axis_name="s")
    return functools.partial(
        pl.kernel, mesh=mesh,
        compiler_params=pltpu.CompilerParams(needs_layout_passes=False),
        out_type=jax.ShapeDtypeStruct((_B,), jnp.float32),
        scratch_types=[
            pltpu.VMEM((4, 128), jnp.int32),
            pltpu.VMEM((4, 128), jnp.int32),
            pltpu.VMEM((_BPW * _D,), jnp.float32),
            pltpu.VMEM((_BPW * _D,), jnp.float32),
            pltpu.VMEM((_BPW,), jnp.float32),
            pltpu.VMEM((_BPW,), jnp.float32),
            pltpu.VMEM((_BPW,), jnp.float32),
            pltpu.SemaphoreType.DMA,
            pltpu.SemaphoreType.DMA,
        ],
    )(_sc_dot_body)


def _tc_cost_body(s_ref, co_ref, out_ref):
    s = s_ref[...]
    co = co_ref[...]
    w = jnp.minimum(1.0, jnp.exp(0.75 * jnp.log(co * (1.0 / 100.0))))
    e = s - jnp.log(co + 1.0)
    out_ref[0, 0] = jnp.sum(w * e * e)


def _tc_cost(s, co):
    out = pl.pallas_call(
        _tc_cost_body,
        out_shape=jax.ShapeDtypeStruct((1, 1), jnp.float32),
        out_specs=pl.BlockSpec(memory_space=pltpu.SMEM),
    )(s.reshape(128, 128), co.reshape(128, 128))
    return out[0, 0]


def kernel(i_ids, j_ids, co_occurs, target_embeddings, context_embeddings,
           target_biases, context_biases):
    ii = i_ids.astype(jnp.int32)
    jj = j_ids.astype(jnp.int32)
    ids2 = jnp.concatenate([ii, jj]).reshape(_NID // 128, 128)
    teT = target_embeddings.T
    ceT = context_embeddings.T
    rows = _sc_sweep()(ids2, teT, ceT,
                       teT[:, _FULL * 128:], ceT[:, _FULL * 128:],
                       jnp.zeros((_D,), jnp.float32))
    i3 = ii.reshape(_NW, 4, 128)
    j3 = jj.reshape(_NW, 4, 128)
    s = _sc_dot()(i3, j3, rows, target_biases, context_biases)
    return _tc_cost(s, co_occurs)


# speedup vs baseline: 2.3072x; 1.1545x over previous
import functools

import jax
import jax.numpy as jnp
from jax import lax
from jax.experimental import pallas as pl
from jax.experimental.pallas import tpu as pltpu
from jax.experimental.pallas import tpu_sc as plsc

_NC = 2
_NS = 16
_NW = _NC * _NS
_L = 16
_D = 64
_B = 16384
_BPW = _B // _NW
_V = 1000000
_NBLK = (_V + 127) // 128
_FULL = _V // 128
_NID = 2 * _B


def _ids_chunk(idv, c):
    return idv[lax.shift_right_logical(c, 3),
               pl.ds(jnp.bitwise_and(c, 7) * _L, _L)]


def _sc_sweep_body(ids_hbm, te_hbm, ce_hbm, tet_hbm, cet_hbm, dum_hbm,
                   out_hbm,
                   idv, hits, counts, offs, cur,
                   sat, sac, sbt, sbc, sct, scc, stag,
                   semA, semB, semC, semw0, semw1, semw2, semw3):
    wid = lax.axis_index("s") * _NC + lax.axis_index("c")
    lo = lax.shift_right_logical(wid * _NBLK, 5)
    hi = lax.shift_right_logical((wid + 1) * _NBLK, 5)
    pltpu.sync_copy(ids_hbm, idv)

    lanes = lax.iota(jnp.int32, _L)
    zeros = jnp.zeros((_L,), jnp.int32)
    ones = jnp.full((_L,), 1, jnp.int32)

    for k in range(256 // _L):
        counts[pl.ds(k * _L, _L)] = zeros

    def hist(c, carry):
        v = _ids_chunk(idv, c)
        vb = lax.shift_right_logical(v, 7)
        mask = jnp.logical_and(vb >= lo, vb < hi)
        rel = jnp.clip(vb - lo, 0, 255)
        plsc.addupdate_scatter(counts, [rel], ones, mask=mask)
        return carry

    lax.fori_loop(0, _NID // _L, hist, 0, unroll=False)

    run = jnp.int32(0)
    for k in range(256 // _L):
        sl = pl.ds(k * _L, _L)
        ch = counts[sl]
        inc = plsc.cumsum(ch)
        offs[sl] = run + inc - ch
        cur[sl] = run + inc - ch
        run = run + inc[15]

    def place(c, carry):
        v = _ids_chunk(idv, c)
        vb = lax.shift_right_logical(v, 7)
        mask = jnp.logical_and(vb >= lo, vb < hi)
        rel = jnp.clip(vb - lo, 0, 255)
        prior = plsc.load_gather(cur, [rel])
        occ = plsc.scan_count(rel, mask=mask)[0]
        b = c * _L + lanes
        payload = jnp.bitwise_or(jnp.bitwise_and(v, 127),
                                 lax.shift_left(b, 7))
        pos = jnp.clip(prior + occ - 1, 0, _NID - 1)
        plsc.store_scatter(hits, [pos], payload, mask=mask)
        plsc.addupdate_scatter(cur, [rel], ones, mask=mask)
        return carry

    lax.fori_loop(0, _NID // _L, place, 0, unroll=False)

    dchunks = [k * _L + lanes for k in range(_D // _L)]

    def extract_hits(rel, tbuf, cbuf, semw_list):
        s0 = plsc.load_gather(offs, [jnp.full((_L,), rel, jnp.int32)])[0]
        n0 = plsc.load_gather(counts, [jnp.full((_L,), rel, jnp.int32)])[0]

        def hit_body(h, carry):
            hv = plsc.load_gather(hits, [jnp.full((_L,), h, jnp.int32)])[0]
            lane = jnp.bitwise_and(hv, 127)
            b = lax.shift_right_logical(hv, 7)
            tb1 = lax.shift_right_logical(b, 14)
            for slot in range(4):
                @pl.when(jnp.bitwise_and(h, 3) == slot)
                def _():
                    @pl.when(h >= 4)
                    def _():
                        pltpu.make_async_copy(
                            dum_hbm, stag.at[pl.ds(0, _D)],
                            semw_list[slot]).wait()
                    ssl = pl.ds(slot * _D, _D)
                    for tsel in range(2):
                        @pl.when(tb1 == tsel)
                        def _():
                            src = tbuf if tsel == 0 else cbuf
                            lanev = jnp.full((_L,), lane, jnp.int32)
                            for k in range(_D // _L):
                                stag[pl.ds(slot * _D + k * _L, _L)] = (
                                    plsc.load_gather(src, [dchunks[k], lanev]))
                    pltpu.async_copy(stag.at[ssl],
                                     out_hbm.at[pl.ds(b * _D, _D)],
                                     semw_list[slot])
            return carry

        lax.fori_loop(s0, s0 + n0, hit_body, 0, unroll=False)

    himain = jnp.minimum(hi, _FULL)

    def fire(vb, tbuf, cbuf, sem):
        sl = pl.ds(vb * 128, 128)
        pltpu.async_copy(te_hbm.at[:, sl], tbuf, sem)
        pltpu.async_copy(ce_hbm.at[:, sl], cbuf, sem)

    def drain(tbuf, cbuf, sem):
        pltpu.make_async_copy(te_hbm.at[:, pl.ds(0, 128)], tbuf, sem).wait()
        pltpu.make_async_copy(ce_hbm.at[:, pl.ds(0, 128)], cbuf, sem).wait()

    ring = ((sat, sac, semA), (sbt, sbc, semB), (sct, scc, semC))
    nring = len(ring)
    for d, (tb, cb, sem) in enumerate(ring):
        @pl.when(lo + d < himain)
        def _():
            fire(lo + d, tb, cb, sem)

    ngroups = lax.div(himain - lo + (nring - 1), nring)

    def group_body(k, carry):
        vbb = lo + nring * k
        for d, (tb, cb, sem) in enumerate(ring):
            vb = vbb + d

            @pl.when(vb < himain)
            def _():
                drain(tb, cb, sem)
                extract_hits(vb - lo, tb, cb, [semw0, semw1, semw2, semw3])

            @pl.when(vb + nring < himain)
            def _():
                fire(vb + nring, tb, cb, sem)

        return carry

    lax.fori_loop(0, ngroups, group_body, 0, unroll=False)

    @pl.when(hi == _NBLK)
    def _():
        pltpu.sync_copy(tet_hbm, sat)
        pltpu.sync_copy(cet_hbm, sac)
        extract_hits(_FULL - lo, sat, sac, [semw0, semw1, semw2, semw3])

    relL = jnp.full((_L,), hi - lo - 1, jnp.int32)
    nh = plsc.load_gather(offs, [relL])[0] + plsc.load_gather(counts, [relL])[0]
    for slot, sem in enumerate((semw0, semw1, semw2, semw3)):
        @pl.when(nh > slot)
        def _():
            pltpu.make_async_copy(dum_hbm, stag.at[pl.ds(0, _D)], sem).wait()


@functools.lru_cache(maxsize=1)
def _sc_sweep():
    mesh = plsc.VectorSubcoreMesh(core_axis_name="c", subcore_axis_name="s")
    return functools.partial(
        pl.kernel, mesh=mesh,
        compiler_params=pltpu.CompilerParams(needs_layout_passes=False),
        out_type=jax.ShapeDtypeStruct((_NID * _D,), jnp.float32),
        scratch_types=[
            pltpu.VMEM((_NID // 128, 128), jnp.int32),
            pltpu.VMEM((_NID,), jnp.int32),
            pltpu.VMEM((256,), jnp.int32),
            pltpu.VMEM((256,), jnp.int32),
            pltpu.VMEM((256,), jnp.int32),
            pltpu.VMEM((_D, 128), jnp.float32),
            pltpu.VMEM((_D, 128), jnp.float32),
            pltpu.VMEM((_D, 128), jnp.float32),
            pltpu.VMEM((_D, 128), jnp.float32),
            pltpu.VMEM((_D, 128), jnp.float32),
            pltpu.VMEM((_D, 128), jnp.float32),
            pltpu.VMEM((4 * _D,), jnp.float32),
            pltpu.SemaphoreType.DMA,
            pltpu.SemaphoreType.DMA,
            pltpu.SemaphoreType.DMA,
            pltpu.SemaphoreType.DMA,
            pltpu.SemaphoreType.DMA,
            pltpu.SemaphoreType.DMA,
            pltpu.SemaphoreType.DMA,
        ],
    )(_sc_sweep_body)


def _sc_dot_body(i_hbm, j_hbm, rows_hbm, tb_hbm, cb_hbm, out_hbm,
                 iv, jv, tv, cv, tbv, cbv, sv, semr, semb):
    wid = lax.axis_index("s") * _NC + lax.axis_index("c")
    base = wid * _BPW
    pltpu.sync_copy(i_hbm.at[wid], iv)
    pltpu.sync_copy(j_hbm.at[wid], jv)

    bias_copies = []
    for blk in range(4):
        r = pl.ds(blk * 128, 128)
        bias_copies.append(pltpu.async_copy(tb_hbm.at[iv.at[blk]], tbv.at[r], semb))
        bias_copies.append(pltpu.async_copy(cb_hbm.at[jv.at[blk]], cbv.at[r], semb))

    ct = pltpu.async_copy(rows_hbm.at[pl.ds(base * _D, _BPW * _D)], tv, semr)
    cc = pltpu.async_copy(
        rows_hbm.at[pl.ds((_B + base) * _D, _BPW * _D)], cv, semr)
    ct.wait()
    cc.wait()
    for c in bias_copies:
        c.wait()

    lanes = lax.iota(jnp.int32, _L)

    def group(g, carry):
        o = g * _L
        svec = jnp.zeros((_L,), jnp.float32)
        for r in range(_L):
            rb = (o + r) * _D
            acc = tv[pl.ds(rb, _L)] * cv[pl.ds(rb, _L)]
            for c in range(1, _D // _L):
                acc = acc + (tv[pl.ds(rb + c * _L, _L)]
                             * cv[pl.ds(rb + c * _L, _L)])
            svec = jnp.where(lanes == r, jnp.sum(acc), svec)
        sl = pl.ds(o, _L)
        sv[sl] = svec + tbv[sl] + cbv[sl]
        return carry

    lax.fori_loop(0, _BPW // _L, group, 0, unroll=False)
    pltpu.sync_copy(sv, out_hbm.at[pl.ds(base, _BPW)])


@functools.lru_cache(maxsize=1)
def _sc_dot():
    mesh = plsc.VectorSubcoreMesh(core_axis_name="c", subcore_axis_name="s")
    return functools.partial(
        pl.kernel, mesh=mesh,
        compiler_params=pltpu.CompilerParams(needs_layout_passes=False),
        out_type=jax.ShapeDtypeStruct((_B,), jnp.float32),
        scratch_types=[
            pltpu.VMEM((4, 128), jnp.int32),
            pltpu.VMEM((4, 128), jnp.int32),
            pltpu.VMEM((_BPW * _D,), jnp.float32),
            pltpu.VMEM((_BPW * _D,), jnp.float32),
            pltpu.VMEM((_BPW,), jnp.float32),
            pltpu.VMEM((_BPW,), jnp.float32),
            pltpu.VMEM((_BPW,), jnp.float32),
            pltpu.SemaphoreType.DMA,
            pltpu.SemaphoreType.DMA,
        ],
    )(_sc_dot_body)


def _tc_cost_body(s_ref, co_ref, out_ref):
    s = s_ref[...]
    co = co_ref[...]
    w = jnp.minimum(1.0, jnp.exp(0.75 * jnp.log(co * (1.0 / 100.0))))
    e = s - jnp.log(co + 1.0)
    out_ref[0, 0] = jnp.sum(w * e * e)


def _tc_cost(s, co):
    out = pl.pallas_call(
        _tc_cost_body,
        out_shape=jax.ShapeDtypeStruct((1, 1), jnp.float32),
        out_specs=pl.BlockSpec(memory_space=pltpu.SMEM),
    )(s.reshape(128, 128), co.reshape(128, 128))
    return out[0, 0]


def kernel(i_ids, j_ids, co_occurs, target_embeddings, context_embeddings,
           target_biases, context_biases):
    ii = i_ids.astype(jnp.int32)
    jj = j_ids.astype(jnp.int32)
    ids2 = jnp.concatenate([ii, jj]).reshape(_NID // 128, 128)
    teT = target_embeddings.T
    ceT = context_embeddings.T
    tail_pad = ((0, 0), (0, 128 - (_V - _FULL * 128)))
    rows = _sc_sweep()(ids2, teT, ceT,
                       jnp.pad(teT[:, _FULL * 128:], tail_pad),
                       jnp.pad(ceT[:, _FULL * 128:], tail_pad),
                       jnp.zeros((_D,), jnp.float32))
    i3 = ii.reshape(_NW, 4, 128)
    j3 = jj.reshape(_NW, 4, 128)
    s = _sc_dot()(i3, j3, rows, target_biases, context_biases)
    return _tc_cost(s, co_occurs)


# speedup vs baseline: 2.3164x; 1.0040x over previous
import functools

import jax
import jax.numpy as jnp
from jax import lax
from jax.experimental import pallas as pl
from jax.experimental.pallas import tpu as pltpu
from jax.experimental.pallas import tpu_sc as plsc

_NC = 2
_NS = 16
_NW = _NC * _NS
_L = 16
_D = 64
_B = 16384
_BPW = _B // _NW
_V = 1000000
_SB = 256
_NBLK = (_V + _SB - 1) // _SB
_FULL = _V // _SB
_NID = 2 * _B


def _ids_chunk(idv, c):
    return idv[lax.shift_right_logical(c, 3),
               pl.ds(jnp.bitwise_and(c, 7) * _L, _L)]


def _sc_sweep_body(ids_hbm, te_hbm, ce_hbm, tet_hbm, cet_hbm, dum_hbm,
                   out_hbm,
                   idv, hits, counts, offs, cur,
                   s0_, s1_, s2_, stag, ec_smem,
                   semA, semB, semC, semw0, semw1, semw2, semw3):
    wid = lax.axis_index("s") * _NC + lax.axis_index("c")
    lo = lax.shift_right_logical(wid * _NBLK, 5)
    hi = lax.shift_right_logical((wid + 1) * _NBLK, 5)
    pltpu.sync_copy(ids_hbm, idv)
    ec_smem[0] = jnp.int32(0)

    lanes = lax.iota(jnp.int32, _L)
    zeros = jnp.zeros((_L,), jnp.int32)
    ones = jnp.full((_L,), 1, jnp.int32)

    for k in range(128 // _L):
        counts[pl.ds(k * _L, _L)] = zeros

    def hist(c, carry):
        v = _ids_chunk(idv, c)
        vb = lax.shift_right_logical(v, 8)
        mask = jnp.logical_and(vb >= lo, vb < hi)
        rel = jnp.clip(vb - lo, 0, 127)
        plsc.addupdate_scatter(counts, [rel], ones, mask=mask)
        return carry

    lax.fori_loop(0, _NID // _L, hist, 0, unroll=False)

    run = jnp.int32(0)
    for k in range(128 // _L):
        sl = pl.ds(k * _L, _L)
        ch = counts[sl]
        inc = plsc.cumsum(ch)
        offs[sl] = run + inc - ch
        cur[sl] = run + inc - ch
        run = run + inc[15]

    def place(c, carry):
        v = _ids_chunk(idv, c)
        vb = lax.shift_right_logical(v, 8)
        mask = jnp.logical_and(vb >= lo, vb < hi)
        rel = jnp.clip(vb - lo, 0, 127)
        prior = plsc.load_gather(cur, [rel])
        occ = plsc.scan_count(rel, mask=mask)[0]
        b = c * _L + lanes
        payload = jnp.bitwise_or(jnp.bitwise_and(v, _SB - 1),
                                 lax.shift_left(b, 8))
        pos = jnp.clip(prior + occ - 1, 0, _NID - 1)
        plsc.store_scatter(hits, [pos], payload, mask=mask)
        plsc.addupdate_scatter(cur, [rel], ones, mask=mask)
        return carry

    lax.fori_loop(0, _NID // _L, place, 0, unroll=False)

    dchunks = [k * _L + lanes for k in range(_D // _L)]
    semw = (semw0, semw1, semw2, semw3)

    def extract_hits(rel, buf, tsel):
        s0 = plsc.load_gather(offs, [jnp.full((_L,), rel, jnp.int32)])[0]
        n0 = plsc.load_gather(counts, [jnp.full((_L,), rel, jnp.int32)])[0]

        def hit_body(h, carry):
            hv = plsc.load_gather(hits, [jnp.full((_L,), h, jnp.int32)])[0]
            lane = jnp.bitwise_and(hv, _SB - 1)
            b = lax.shift_right_logical(hv, 8)
            tb1 = lax.shift_right_logical(b, 14)

            @pl.when(tb1 == tsel)
            def _():
                ec = ec_smem[0]
                for slot in range(4):
                    @pl.when(jnp.bitwise_and(ec, 3) == slot)
                    def _():
                        @pl.when(ec >= 4)
                        def _():
                            pltpu.make_async_copy(
                                dum_hbm, stag.at[pl.ds(0, _D)],
                                semw[slot]).wait()
                        lanev = jnp.full((_L,), lane, jnp.int32)
                        for k in range(_D // _L):
                            stag[pl.ds(slot * _D + k * _L, _L)] = (
                                plsc.load_gather(buf, [dchunks[k], lanev]))
                        pltpu.async_copy(stag.at[pl.ds(slot * _D, _D)],
                                         out_hbm.at[pl.ds(b * _D, _D)],
                                         semw[slot])
                ec_smem[0] = ec + 1
            return carry

        lax.fori_loop(s0, s0 + n0, hit_body, 0, unroll=False)

    himain = jnp.minimum(hi, _FULL)
    ring = ((s0_, semA), (s1_, semB), (s2_, semC))
    nring = len(ring)

    def sweep(src_hbm, tail_hbm, tsel):
        def fire(vb, buf, sem):
            pltpu.async_copy(src_hbm.at[:, pl.ds(vb * _SB, _SB)], buf, sem)

        def drain(buf, sem):
            pltpu.make_async_copy(src_hbm.at[:, pl.ds(0, _SB)], buf,
                                  sem).wait()

        for d, (buf, sem) in enumerate(ring):
            @pl.when(lo + d < himain)
            def _():
                fire(lo + d, buf, sem)

        ngroups = lax.div(himain - lo + (nring - 1), nring)

        def group_body(k, carry):
            vbb = lo + nring * k
            for d, (buf, sem) in enumerate(ring):
                vb = vbb + d

                @pl.when(vb < himain)
                def _():
                    drain(buf, sem)
                    extract_hits(vb - lo, buf, tsel)

                @pl.when(vb + nring < himain)
                def _():
                    fire(vb + nring, buf, sem)

            return carry

        lax.fori_loop(0, ngroups, group_body, 0, unroll=False)

        @pl.when(hi == _NBLK)
        def _():
            pltpu.sync_copy(tail_hbm, s0_)
            extract_hits(_FULL - lo, s0_, tsel)

    sweep(te_hbm, tet_hbm, 0)
    sweep(ce_hbm, cet_hbm, 1)

    nh = ec_smem[0]
    for slot in range(4):
        @pl.when(nh > slot)
        def _():
            pltpu.make_async_copy(dum_hbm, stag.at[pl.ds(0, _D)],
                                  semw[slot]).wait()


@functools.lru_cache(maxsize=1)
def _sc_sweep():
    mesh = plsc.VectorSubcoreMesh(core_axis_name="c", subcore_axis_name="s")
    return functools.partial(
        pl.kernel, mesh=mesh,
        compiler_params=pltpu.CompilerParams(needs_layout_passes=False),
        out_type=jax.ShapeDtypeStruct((_NID * _D,), jnp.float32),
        scratch_types=[
            pltpu.VMEM((_NID // 128, 128), jnp.int32),
            pltpu.VMEM((_NID,), jnp.int32),
            pltpu.VMEM((128,), jnp.int32),
            pltpu.VMEM((128,), jnp.int32),
            pltpu.VMEM((128,), jnp.int32),
            pltpu.VMEM((_D, _SB), jnp.float32),
            pltpu.VMEM((_D, _SB), jnp.float32),
            pltpu.VMEM((_D, _SB), jnp.float32),
            pltpu.VMEM((4 * _D,), jnp.float32),
            pltpu.SMEM((8,), jnp.int32),
            pltpu.SemaphoreType.DMA,
            pltpu.SemaphoreType.DMA,
            pltpu.SemaphoreType.DMA,
            pltpu.SemaphoreType.DMA,
            pltpu.SemaphoreType.DMA,
            pltpu.SemaphoreType.DMA,
            pltpu.SemaphoreType.DMA,
        ],
    )(_sc_sweep_body)


def _sc_dot_body(i_hbm, j_hbm, rows_hbm, tb_hbm, cb_hbm, out_hbm,
                 iv, jv, tv, cv, tbv, cbv, sv, semr, semb):
    wid = lax.axis_index("s") * _NC + lax.axis_index("c")
    base = wid * _BPW
    pltpu.sync_copy(i_hbm.at[wid], iv)
    pltpu.sync_copy(j_hbm.at[wid], jv)

    bias_copies = []
    for blk in range(4):
        r = pl.ds(blk * 128, 128)
        bias_copies.append(pltpu.async_copy(tb_hbm.at[iv.at[blk]], tbv.at[r], semb))
        bias_copies.append(pltpu.async_copy(cb_hbm.at[jv.at[blk]], cbv.at[r], semb))

    ct = pltpu.async_copy(rows_hbm.at[pl.ds(base * _D, _BPW * _D)], tv, semr)
    cc = pltpu.async_copy(
        rows_hbm.at[pl.ds((_B + base) * _D, _BPW * _D)], cv, semr)
    ct.wait()
    cc.wait()
    for c in bias_copies:
        c.wait()

    lanes = lax.iota(jnp.int32, _L)

    def group(g, carry):
        o = g * _L
        svec = jnp.zeros((_L,), jnp.float32)
        for r in range(_L):
            rb = (o + r) * _D
            acc = tv[pl.ds(rb, _L)] * cv[pl.ds(rb, _L)]
            for c in range(1, _D // _L):
                acc = acc + (tv[pl.ds(rb + c * _L, _L)]
                             * cv[pl.ds(rb + c * _L, _L)])
            svec = jnp.where(lanes == r, jnp.sum(acc), svec)
        sl = pl.ds(o, _L)
        sv[sl] = svec + tbv[sl] + cbv[sl]
        return carry

    lax.fori_loop(0, _BPW // _L, group, 0, unroll=False)
    pltpu.sync_copy(sv, out_hbm.at[pl.ds(base, _BPW)])


@functools.lru_cache(maxsize=1)
def _sc_dot():
    mesh = plsc.VectorSubcoreMesh(core_axis_name="c", subcore_axis_name="s")
    return functools.partial(
        pl.kernel, mesh=mesh,
        compiler_params=pltpu.CompilerParams(needs_layout_passes=False),
        out_type=jax.ShapeDtypeStruct((_B,), jnp.float32),
        scratch_types=[
            pltpu.VMEM((4, 128), jnp.int32),
            pltpu.VMEM((4, 128), jnp.int32),
            pltpu.VMEM((_BPW * _D,), jnp.float32),
            pltpu.VMEM((_BPW * _D,), jnp.float32),
            pltpu.VMEM((_BPW,), jnp.float32),
            pltpu.VMEM((_BPW,), jnp.float32),
            pltpu.VMEM((_BPW,), jnp.float32),
            pltpu.SemaphoreType.DMA,
            pltpu.SemaphoreType.DMA,
        ],
    )(_sc_dot_body)


def _tc_cost_body(s_ref, co_ref, out_ref):
    s = s_ref[...]
    co = co_ref[...]
    w = jnp.minimum(1.0, jnp.exp(0.75 * jnp.log(co * (1.0 / 100.0))))
    e = s - jnp.log(co + 1.0)
    out_ref[0, 0] = jnp.sum(w * e * e)


def _tc_cost(s, co):
    out = pl.pallas_call(
        _tc_cost_body,
        out_shape=jax.ShapeDtypeStruct((1, 1), jnp.float32),
        out_specs=pl.BlockSpec(memory_space=pltpu.SMEM),
    )(s.reshape(128, 128), co.reshape(128, 128))
    return out[0, 0]


def kernel(i_ids, j_ids, co_occurs, target_embeddings, context_embeddings,
           target_biases, context_biases):
    ii = i_ids.astype(jnp.int32)
    jj = j_ids.astype(jnp.int32)
    ids2 = jnp.concatenate([ii, jj]).reshape(_NID // 128, 128)
    teT = target_embeddings.T
    ceT = context_embeddings.T
    tail_pad = ((0, 0), (0, _SB - (_V - _FULL * _SB)))
    rows = _sc_sweep()(ids2, teT, ceT,
                       jnp.pad(teT[:, _FULL * _SB:], tail_pad),
                       jnp.pad(ceT[:, _FULL * _SB:], tail_pad),
                       jnp.zeros((_D,), jnp.float32))
    i3 = ii.reshape(_NW, 4, 128)
    j3 = jj.reshape(_NW, 4, 128)
    s = _sc_dot()(i3, j3, rows, target_biases, context_biases)
    return _tc_cost(s, co_occurs)
